# baseline (speedup 1.0000x reference)
import functools

import jax
import jax.numpy as jnp
from jax import lax
from jax.experimental import pallas as pl
from jax.experimental.pallas import tpu as pltpu
from jax.experimental.pallas import tpu_sc as plsc

_EPS = 1e-5
_BOUNDARY = 1.0 - _EPS
_VOCAB = 1000000
_DIM = 32
_BATCH = 16384
_SAMPLES = 50

_NC, _NS = 2, 16
_NW = _NC * _NS
_BPW = _BATCH // _NW
_CB = 16
_NCH = _BPW // _CB


def _sc_gather_body(table_hbm, idx_hbm, out_hbm, idx_v, rows_v, sem):
    wid = lax.axis_index("s") * _NC + lax.axis_index("c")
    base = wid * _BPW
    for c in range(_NCH):
        b0 = base + c * _CB
        pltpu.sync_copy(idx_hbm.at[pl.ds(b0, _CB)], idx_v)
        descs = [
            pltpu.async_copy(table_hbm.at[idx_v.at[b]], rows_v.at[b], sem)
            for b in range(_CB)
        ]
        for d in descs:
            d.wait()
        pltpu.sync_copy(rows_v, out_hbm.at[pl.ds(b0, _CB)])


@functools.cache
def _sc_gather():
    return pl.kernel(
        _sc_gather_body,
        out_type=jax.ShapeDtypeStruct((_BATCH, _SAMPLES, _DIM), jnp.float32),
        mesh=plsc.VectorSubcoreMesh(
            core_axis_name="c", subcore_axis_name="s",
            num_cores=_NC, num_subcores=_NS,
        ),
        scratch_types=[
            pltpu.VMEM((_CB, _SAMPLES), jnp.int32),
            pltpu.VMEM((_CB, _SAMPLES, _DIM), jnp.float32),
            pltpu.SemaphoreType.DMA,
        ],
        compiler_params=pltpu.CompilerParams(use_tc_tiling_on_sc=False),
    )


_BB = 128


def _dist_body(e_ref, out_ref):
    e = e_ref[...]
    n = jnp.sqrt(jnp.sum(e * e, axis=-1, keepdims=True))
    scale = jnp.where(n > 1.0, 1.0 / (n + 1e-7), 1.0)
    e = e * scale
    sq = jnp.clip(jnp.sum(e * e, axis=-1), 0.0, _BOUNDARY)
    u = e[:, :1, :]
    o = e[:, 1:, :]
    sqdist = jnp.sum(jnp.square(u - o), axis=-1)
    squ = sq[:, :1]
    sqv = sq[:, 1:]
    x = sqdist / ((1.0 - squ) * (1.0 - sqv)) * 2.0 + 1.0
    z = jnp.sqrt(jnp.maximum(x * x - 1.0, 1e-12))
    out_ref[...] = -jnp.log(x + z)


_dist = pl.pallas_call(
    _dist_body,
    grid=(_BATCH // _BB,),
    in_specs=[pl.BlockSpec((_BB, _SAMPLES, _DIM), lambda i: (i, 0, 0))],
    out_specs=pl.BlockSpec((_BB, _SAMPLES - 1), lambda i: (i, 0)),
    out_shape=jax.ShapeDtypeStruct((_BATCH, _SAMPLES - 1), jnp.float32),
)


def kernel(inputs, weight):
    e = _sc_gather()(weight, inputs)
    return _dist(e)


# speedup vs baseline: 1.4040x; 1.4040x over previous
import functools

import jax
import jax.numpy as jnp
from jax import lax
from jax.experimental import pallas as pl
from jax.experimental.pallas import tpu as pltpu
from jax.experimental.pallas import tpu_sc as plsc

_EPS = 1e-5
_BOUNDARY = 1.0 - _EPS
_VOCAB = 1000000
_DIM = 32
_BATCH = 16384
_SAMPLES = 50

_NC, _NS = 2, 16
_NW = _NC * _NS
_BPW = _BATCH // _NW
_CB = 16
_NCH = _BPW // _CB
_HL = _DIM // 2


def _sc_body(table_hbm, idx_hbm, n2_hbm, dot_hbm,
             idx_v, rows_v, n2_v, dot_v, sem):
    wid = lax.axis_index("s") * _NC + lax.axis_index("c")
    base = wid * _BPW
    lanes = lax.iota(jnp.int32, 16)
    zero16 = jnp.zeros((16,), jnp.float32)

    def cbody(c, carry):
        b0 = base + c * _CB
        pltpu.sync_copy(idx_hbm.at[pl.ds(b0, _CB)], idx_v)
        descs = [
            pltpu.async_copy(
                table_hbm.at[idx_v.at[b]],
                rows_v.at[pl.ds(b * _SAMPLES, _SAMPLES)], sem)
            for b in range(_CB)
        ]
        for d in descs:
            d.wait()

        def bbody(b, carry):
            rb = b * _SAMPLES
            bvec = lanes * 0 + b
            for g in range(4):
                svec = jnp.minimum(lanes + 16 * g, _SAMPLES - 1)
                rvec = svec + rb

                def dbody(d, acc, rvec=rvec, rbvec=lanes * 0 + rb):
                    n2a, dta = acc
                    dvec = lanes * 0 + d
                    v = plsc.load_gather(rows_v, [rvec, dvec])
                    u_d = plsc.load_gather(rows_v, [rbvec, dvec])
                    return (n2a + v * v, dta + u_d * v)

                n2a, dta = lax.fori_loop(
                    0, _DIM, dbody, (zero16, zero16), unroll=8)
                plsc.store_scatter(n2_v, [bvec, svec], n2a)
                plsc.store_scatter(
                    dot_v, [bvec, svec - 1], dta, mask=svec >= 1)
            return carry

        lax.fori_loop(0, _CB, bbody, 0)
        pltpu.sync_copy(n2_v, n2_hbm.at[pl.ds(b0, _CB)])
        pltpu.sync_copy(dot_v, dot_hbm.at[pl.ds(b0, _CB)])
        return carry

    lax.fori_loop(0, _NCH, cbody, 0)


@functools.cache
def _sc_gather_reduce():
    return pl.kernel(
        _sc_body,
        out_type=(
            jax.ShapeDtypeStruct((_BATCH, _SAMPLES), jnp.float32),
            jax.ShapeDtypeStruct((_BATCH, _SAMPLES - 1), jnp.float32),
        ),
        mesh=plsc.VectorSubcoreMesh(
            core_axis_name="c", subcore_axis_name="s",
            num_cores=_NC, num_subcores=_NS,
        ),
        scratch_types=[
            pltpu.VMEM((_CB, _SAMPLES), jnp.int32),
            pltpu.VMEM((_CB * _SAMPLES, _DIM), jnp.float32),
            pltpu.VMEM((_CB, _SAMPLES), jnp.float32),
            pltpu.VMEM((_CB, _SAMPLES - 1), jnp.float32),
            pltpu.SemaphoreType.DMA,
        ],
        compiler_params=pltpu.CompilerParams(
            use_tc_tiling_on_sc=False, needs_layout_passes=False),
    )


_BB = 2048


def _fin_body(n2_ref, dot_ref, out_ref):
    n2 = n2_ref[...]
    dt = dot_ref[...]
    n = jnp.sqrt(n2)
    scale = jnp.where(n > 1.0, 1.0 / (n + 1e-7), 1.0)
    sq_raw = n2 * scale * scale
    sq = jnp.clip(sq_raw, 0.0, _BOUNDARY)
    u2 = sq_raw[:, :1]
    v2 = sq_raw[:, 1:]
    su = scale[:, :1]
    sv = scale[:, 1:]
    sqdist = u2 + v2 - 2.0 * (su * sv) * dt
    squ = sq[:, :1]
    sqv = sq[:, 1:]
    x = sqdist / ((1.0 - squ) * (1.0 - sqv)) * 2.0 + 1.0
    z = jnp.sqrt(jnp.maximum(x * x - 1.0, 1e-12))
    out_ref[...] = -jnp.log(x + z)


_fin = pl.pallas_call(
    _fin_body,
    grid=(_BATCH // _BB,),
    in_specs=[
        pl.BlockSpec((_BB, _SAMPLES), lambda i: (i, 0)),
        pl.BlockSpec((_BB, _SAMPLES - 1), lambda i: (i, 0)),
    ],
    out_specs=pl.BlockSpec((_BB, _SAMPLES - 1), lambda i: (i, 0)),
    out_shape=jax.ShapeDtypeStruct((_BATCH, _SAMPLES - 1), jnp.float32),
)


def kernel(inputs, weight):
    n2, dt = _sc_gather_reduce()(weight, inputs)
    return _fin(n2, dt)
